# baseline (speedup 1.0000x reference)
import functools

import jax
import jax.numpy as jnp
from jax import lax
from jax.experimental import pallas as pl
from jax.experimental.pallas import tpu as pltpu
from jax.experimental.pallas import tpu_sc as plsc

N = 10000
D = 128
HALF = 5120
NR = 6144
RPT = 384
EPT = 20480
CH = 128
NCH = EPT // CH
SB = 32
NB = NCH // SB
GDR = 80
DDR = NR // CH


def _prep_body(x_ref, wlin_ref, wconv_ref, bconv_ref, war_ref,
               pg_ref, hb1_ref, b0_ref, g_ref):
    x = x_ref[...]
    h = lax.dot_general(x, wlin_ref[...], (((1,), (1,)), ((), ())),
                        preferred_element_type=jnp.float32)
    sr = lax.dot_general(h, war_ref[...], (((1,), (1,)), ((), ())),
                         preferred_element_type=jnp.float32)
    g = jnp.exp(sr - jnp.max(sr))
    logits = lax.dot_general(x, wconv_ref[...], (((1,), (1,)), ((), ())),
                             preferred_element_type=jnp.float32)
    logits = logits + bconv_ref[...]
    m = jnp.max(logits, axis=1, keepdims=True)
    eb = jnp.exp(logits - m)
    beta = eb / jnp.sum(eb, axis=1, keepdims=True)
    pg_ref[...] = h * g
    hb1_ref[...] = h * beta[:, 1:2]
    b0_ref[...] = beta[:, 0:1]
    g_ref[...] = g


def _combine_body(parts_ref, hb1_ref, b0_ref, den_ref, out_ref):
    agg = jnp.concatenate(
        [parts_ref[0][:HALF, :], parts_ref[1][:N - HALF, :]], axis=0)
    scale = b0_ref[...] / (den_ref[...] + 1e-30)
    out_ref[...] = agg * scale + hb1_ref[...]


_edge_mesh = plsc.VectorSubcoreMesh(core_axis_name="c", subcore_axis_name="s")


def _den_update(den_ref, d16, gv):
    io16 = lax.broadcasted_iota(jnp.int32, (16,), 0)
    k, v = plsc.sort_key_val(d16, gv)
    csum = plsc.cumsum(v)
    kprev = jnp.take_along_axis(k, jnp.maximum(io16 - 1, 0), axis=0)
    starts = jnp.logical_or(k != kprev, io16 == 0)
    knext = jnp.take_along_axis(k, jnp.minimum(io16 + 1, 15), axis=0)
    ends = jnp.logical_or(k != knext, io16 == 15)
    sidx = plsc.cummax(jnp.where(starts, io16, 0))
    cz = jnp.where(io16 == 0, 0.0,
                   jnp.take_along_axis(csum, jnp.maximum(io16 - 1, 0), axis=0))
    pc = jnp.take_along_axis(cz, sidx, axis=0)
    plsc.addupdate_scatter(den_ref,
                           [lax.shift_right_logical(k, 7),
                            lax.bitwise_and(k, 127)], csum - pc, mask=ends)


@functools.partial(
    pl.kernel,
    mesh=_edge_mesh,
    compiler_params=pltpu.CompilerParams(needs_layout_passes=False),
    out_type=[jax.ShapeDtypeStruct((2, NR, D), jnp.float32),
              jax.ShapeDtypeStruct((2, DDR, CH), jnp.float32)],
    scratch_types=[
        pltpu.VMEM((3, SB + 2, CH), jnp.int32),
        pltpu.VMEM((3, SB, CH), jnp.int32),
        pltpu.VMEM((CH, D), jnp.float32),
        pltpu.VMEM((CH, D), jnp.float32),
        pltpu.VMEM((GDR, CH), jnp.float32),
        pltpu.VMEM((DDR, CH), jnp.float32),
        pltpu.VMEM((1, DDR), jnp.int32),
        pltpu.VMEM_SHARED((NR, D), jnp.float32),
        pltpu.VMEM_SHARED((DDR, CH), jnp.float32),
        pltpu.SemaphoreType.DMA,
        pltpu.SemaphoreType.DMA,
        pltpu.SemaphoreType.DMA,
    ],
)
def _edge_kernel(pg_hbm, src_hbm, dstl_hbm, g_hbm, zrow_hbm, iden_hbm,
                 out_hbm, outden_hbm,
                 src_v, dst_v, rows_a, rows_b, g_v, den_v, iden_v,
                 accum, den_sh, sem_a, sem_b, sem_i):
    c = lax.axis_index("c")
    s = lax.axis_index("s")

    pltpu.sync_copy(src_hbm.at[s].at[0], src_v.at[0])
    pltpu.sync_copy(dstl_hbm.at[c].at[s].at[0], dst_v.at[0])
    pltpu.sync_copy(g_hbm, g_v)
    pltpu.sync_copy(iden_hbm, iden_v)

    pltpu.sync_copy(zrow_hbm.at[pl.ds(0, DDR)], den_v)
    pltpu.sync_copy(zrow_hbm, rows_a)
    for k in range(RPT // CH):
        pltpu.sync_copy(rows_a, accum.at[pl.ds(s * RPT + k * CH, CH)])

    @pl.when(s == 0)
    def _():
        pltpu.sync_copy(den_v, den_sh)

    plsc.subcore_barrier()

    pltpu.async_copy(pg_hbm.at[src_v.at[0].at[0]], rows_a, sem_a)
    pltpu.async_copy(pg_hbm.at[src_v.at[0].at[1]], rows_b, sem_b)

    for b in range(NB):
        sv = src_v.at[b % 3]
        dv = dst_v.at[b % 3]
        if b >= 1:
            pltpu.make_async_copy(src_hbm.at[s].at[b], sv, sem_i).wait()
            pltpu.make_async_copy(dstl_hbm.at[c].at[s].at[b], dv, sem_i).wait()
        if b + 1 < NB:
            nxt = (b + 1) % 3
            pltpu.async_copy(src_hbm.at[s].at[b + 1], src_v.at[nxt], sem_i)
            pltpu.async_copy(dstl_hbm.at[c].at[s].at[b + 1], dst_v.at[nxt],
                             sem_i)

        def _den_chunk(j, sv=sv, dv=dv):
            for t in range(CH // 16):
                s16 = sv[j, pl.ds(t * 16, 16)]
                d16 = dv[j, pl.ds(t * 16, 16)]
                gv = plsc.load_gather(g_v,
                                      [lax.shift_right_logical(s16, 7),
                                       lax.bitwise_and(s16, 127)])
                _den_update(den_v, d16, gv)

        def row_body(i, carry, sv=sv, dv=dv):
            j = i * 2
            pltpu.make_async_copy(pg_hbm.at[sv.at[j]], rows_a, sem_a).wait()
            pltpu.sync_copy(rows_a, accum.at[dv.at[j]], add=True)
            pltpu.async_copy(pg_hbm.at[sv.at[j + 2]], rows_a, sem_a)
            _den_chunk(j)
            pltpu.make_async_copy(pg_hbm.at[sv.at[j + 1]], rows_b, sem_b).wait()
            pltpu.sync_copy(rows_b, accum.at[dv.at[j + 1]], add=True)
            pltpu.async_copy(pg_hbm.at[sv.at[j + 3]], rows_b, sem_b)
            _den_chunk(j + 1)
            return carry

        lax.fori_loop(0, SB // 2, row_body, 0)

    last = src_v.at[(NB - 1) % 3]
    pltpu.make_async_copy(pg_hbm.at[last.at[SB]], rows_a, sem_a).wait()
    pltpu.make_async_copy(pg_hbm.at[last.at[SB + 1]], rows_b, sem_b).wait()

    pltpu.sync_copy(den_v, den_sh.at[iden_v.at[0]], add=True)
    plsc.subcore_barrier()

    for k in range(RPT // CH):
        r0 = s * RPT + k * CH
        pltpu.sync_copy(accum.at[pl.ds(r0, CH)], rows_a)
        pltpu.sync_copy(rows_a, out_hbm.at[c].at[pl.ds(r0, CH)])

    @pl.when(s == 0)
    def _():
        pltpu.sync_copy(den_sh, den_v)
        pltpu.sync_copy(den_v, outden_hbm.at[c])


def kernel(x_n0, x_index_n0, edge_index_n0_to_n0, W_lin, W_conv, b_conv,
           w_al, b_al, w_ar, b_ar):
    del x_index_n0, w_al, b_al, b_ar
    pg, hb1, b0, g = pl.pallas_call(
        _prep_body,
        out_shape=[jax.ShapeDtypeStruct((N, D), jnp.float32),
                   jax.ShapeDtypeStruct((N, D), jnp.float32),
                   jax.ShapeDtypeStruct((N, 1), jnp.float32),
                   jax.ShapeDtypeStruct((N, 1), jnp.float32)],
    )(x_n0, W_lin, W_conv, b_conv.reshape(1, 2), w_ar)

    dst = edge_index_n0_to_n0[0].astype(jnp.int32)
    src = edge_index_n0_to_n0[1].astype(jnp.int32)
    e = src.shape[0]
    epad = 16 * EPT
    npad = epad - e
    pad_iota = jnp.arange(npad, dtype=jnp.int32)
    src_f = jnp.concatenate([src, pad_iota % jnp.int32(N)])
    dst_f = jnp.concatenate(
        [dst, jnp.int32(N) + pad_iota % jnp.int32(2 * HALF - N)])
    spread = jnp.int32(HALF) + (dst_f % jnp.int32(NR - HALF))
    loc0 = jnp.where(dst_f < HALF, dst_f, spread)
    loc1 = jnp.where(dst_f >= HALF, dst_f - jnp.int32(HALF), spread)
    sblk = src_f.reshape(16, NB, SB, CH)
    look = jnp.concatenate(
        [sblk[:, 1:, :2, :], jnp.zeros((16, 1, 2, CH), jnp.int32)], axis=1)
    src_p = jnp.concatenate([sblk, look], axis=2)
    dstl = jnp.stack([loc0, loc1]).reshape(2, 16, NB, SB, CH)
    gflat = jnp.concatenate(
        [g.reshape(N), jnp.zeros((GDR * CH - N,), jnp.float32)]
    ).reshape(GDR, CH)
    zrow = jnp.zeros((CH, CH), jnp.float32)
    iden = jnp.arange(DDR, dtype=jnp.int32).reshape(1, DDR)

    parts, denp = _edge_kernel(pg, src_p, dstl, gflat, zrow, iden)

    den = jnp.concatenate(
        [denp[0].reshape(NR)[:HALF], denp[1].reshape(NR)[:N - HALF]]
    ).reshape(N, 1)

    out = pl.pallas_call(
        _combine_body,
        out_shape=jax.ShapeDtypeStruct((N, D), jnp.float32),
    )(parts, hb1, b0, den)
    return out


# speedup vs baseline: 1.4865x; 1.4865x over previous
import functools

import jax
import jax.numpy as jnp
from jax import lax
from jax.experimental import pallas as pl
from jax.experimental.pallas import tpu as pltpu
from jax.experimental.pallas import tpu_sc as plsc

N = 10000
D = 128
HALF = 5120
NR = 6144
RPT = 384
EPT = 20480
CH = 128
NCH = EPT // CH
SB = 32
NB = NCH // SB
CMAX = EPT + CH
GDR = 80
DDR = NR // CH


def _prep_body(x_ref, wlin_ref, wconv_ref, bconv_ref, war_ref,
               pg_ref, hb1_ref, b0_ref, g_ref):
    x = x_ref[...]
    h = lax.dot_general(x, wlin_ref[...], (((1,), (1,)), ((), ())),
                        preferred_element_type=jnp.float32)
    sr = lax.dot_general(h, war_ref[...], (((1,), (1,)), ((), ())),
                         preferred_element_type=jnp.float32)
    g = jnp.exp(sr - jnp.max(sr))
    logits = lax.dot_general(x, wconv_ref[...], (((1,), (1,)), ((), ())),
                             preferred_element_type=jnp.float32)
    logits = logits + bconv_ref[...]
    m = jnp.max(logits, axis=1, keepdims=True)
    eb = jnp.exp(logits - m)
    beta = eb / jnp.sum(eb, axis=1, keepdims=True)
    pg_ref[...] = h * g
    hb1_ref[...] = h * beta[:, 1:2]
    b0_ref[...] = beta[:, 0:1]
    g_ref[...] = g


def _combine_body(parts_ref, hb1_ref, b0_ref, den_ref, out_ref):
    agg = jnp.concatenate(
        [parts_ref[0][:HALF, :], parts_ref[1][:N - HALF, :]], axis=0)
    scale = b0_ref[...] / (den_ref[...] + 1e-30)
    out_ref[...] = agg * scale + hb1_ref[...]


_edge_mesh = plsc.VectorSubcoreMesh(core_axis_name="c", subcore_axis_name="s")


def _den_update(den_ref, d16, gv):
    io16 = lax.broadcasted_iota(jnp.int32, (16,), 0)
    k, v = plsc.sort_key_val(d16, gv)
    csum = plsc.cumsum(v)
    kprev = jnp.take_along_axis(k, jnp.maximum(io16 - 1, 0), axis=0)
    starts = jnp.logical_or(k != kprev, io16 == 0)
    knext = jnp.take_along_axis(k, jnp.minimum(io16 + 1, 15), axis=0)
    ends = jnp.logical_or(k != knext, io16 == 15)
    sidx = plsc.cummax(jnp.where(starts, io16, 0))
    cz = jnp.where(io16 == 0, 0.0,
                   jnp.take_along_axis(csum, jnp.maximum(io16 - 1, 0), axis=0))
    pc = jnp.take_along_axis(cz, sidx, axis=0)
    plsc.addupdate_scatter(den_ref,
                           [lax.shift_right_logical(k, 7),
                            lax.bitwise_and(k, 127)], csum - pc, mask=ends)


@functools.partial(
    pl.kernel,
    mesh=_edge_mesh,
    compiler_params=pltpu.CompilerParams(needs_layout_passes=False),
    out_type=[jax.ShapeDtypeStruct((2, NR, D), jnp.float32),
              jax.ShapeDtypeStruct((2, DDR, CH), jnp.float32)],
    scratch_types=[
        pltpu.VMEM((2, SB, CH), jnp.int32),
        pltpu.VMEM((2, SB, CH), jnp.int32),
        pltpu.VMEM((CMAX,), jnp.int32),
        pltpu.VMEM((1, CH), jnp.int32),
        pltpu.VMEM((1, CH), jnp.int32),
        pltpu.VMEM((CH, D), jnp.float32),
        pltpu.VMEM((GDR, CH), jnp.float32),
        pltpu.VMEM((DDR, CH), jnp.float32),
        pltpu.VMEM((1, DDR), jnp.int32),
        pltpu.VMEM_SHARED((NR, D), jnp.float32),
        pltpu.VMEM_SHARED((DDR, CH), jnp.float32),
        pltpu.SemaphoreType.DMA,
        pltpu.SemaphoreType.DMA,
    ],
)
def _edge_kernel(pg_hbm, src_hbm, dstl_hbm, g_hbm, zrow_hbm, iden_hbm,
                 out_hbm, outden_hbm,
                 src_v, dst_v, cpk_v, sidx_v, didx_v, rows_v, g_v, den_v,
                 iden_v, accum, den_sh, sem_a, sem_i):
    c = lax.axis_index("c")
    s = lax.axis_index("s")

    pltpu.sync_copy(src_hbm.at[s].at[0], src_v.at[0])
    pltpu.sync_copy(dstl_hbm.at[c].at[s].at[0], dst_v.at[0])
    pltpu.sync_copy(g_hbm, g_v)
    pltpu.sync_copy(iden_hbm, iden_v)

    pltpu.sync_copy(zrow_hbm.at[pl.ds(0, DDR)], den_v)
    pltpu.sync_copy(zrow_hbm, rows_v)
    for k in range(RPT // CH):
        pltpu.sync_copy(rows_v, accum.at[pl.ds(s * RPT + k * CH, CH)])

    @pl.when(s == 0)
    def _():
        pltpu.sync_copy(den_v, den_sh)

    plsc.subcore_barrier()

    io16 = lax.broadcasted_iota(jnp.int32, (16,), 0)
    off = jnp.int32(0)
    for b in range(NB):
        sv = src_v.at[b % 2]
        dv = dst_v.at[b % 2]
        if b >= 1:
            pltpu.make_async_copy(src_hbm.at[s].at[b], sv, sem_i).wait()
            pltpu.make_async_copy(dstl_hbm.at[c].at[s].at[b], dv, sem_i).wait()
        if b + 1 < NB:
            nxt = (b + 1) % 2
            pltpu.async_copy(src_hbm.at[s].at[b + 1], src_v.at[nxt], sem_i)
            pltpu.async_copy(dstl_hbm.at[c].at[s].at[b + 1], dst_v.at[nxt],
                             sem_i)

        def scan_body(j, off, sv=sv, dv=dv):
            for t in range(CH // 16):
                s16 = sv[j, pl.ds(t * 16, 16)]
                d16 = dv[j, pl.ds(t * 16, 16)]
                m = d16 < HALF
                pk = lax.bitwise_or(lax.shift_left(d16, 16), s16)
                plsc.store_compressed(cpk_v.at[pl.ds(off, 16)], pk, mask=m)
                cnt = jnp.max(plsc.all_reduce_population_count(m))
                off = off + cnt
            return off

        off = lax.fori_loop(0, SB, scan_body, off)

    for t in range(CH // 16):
        cpk_v[pl.ds(off + t * 16, 16)] = lax.shift_left(
            jnp.int32(HALF) + t * 16 + io16, 16)

    nch = lax.shift_right_logical(off + (CH - 1), 7)

    def row_body(j, carry):
        for t in range(CH // 16):
            pk = cpk_v[pl.ds(j * CH + t * 16, 16)]
            s16 = lax.bitwise_and(pk, 0xFFFF)
            d16 = lax.shift_right_logical(pk, 16)
            sidx_v[0, pl.ds(t * 16, 16)] = s16
            didx_v[0, pl.ds(t * 16, 16)] = d16
            gv = plsc.load_gather(g_v, [lax.shift_right_logical(s16, 7),
                                        lax.bitwise_and(s16, 127)])
            _den_update(den_v, d16, gv)
        pltpu.async_copy(pg_hbm.at[sidx_v.at[0]], rows_v, sem_a).wait()
        pltpu.sync_copy(rows_v, accum.at[didx_v.at[0]], add=True)
        return carry

    lax.fori_loop(0, nch, row_body, 0)

    pltpu.sync_copy(den_v, den_sh.at[iden_v.at[0]], add=True)
    plsc.subcore_barrier()

    for k in range(RPT // CH):
        r0 = s * RPT + k * CH
        pltpu.sync_copy(accum.at[pl.ds(r0, CH)], rows_v)
        pltpu.sync_copy(rows_v, out_hbm.at[c].at[pl.ds(r0, CH)])

    @pl.when(s == 0)
    def _():
        pltpu.sync_copy(den_sh, den_v)
        pltpu.sync_copy(den_v, outden_hbm.at[c])


def kernel(x_n0, x_index_n0, edge_index_n0_to_n0, W_lin, W_conv, b_conv,
           w_al, b_al, w_ar, b_ar):
    del x_index_n0, w_al, b_al, b_ar
    pg, hb1, b0, g = pl.pallas_call(
        _prep_body,
        out_shape=[jax.ShapeDtypeStruct((N, D), jnp.float32),
                   jax.ShapeDtypeStruct((N, D), jnp.float32),
                   jax.ShapeDtypeStruct((N, 1), jnp.float32),
                   jax.ShapeDtypeStruct((N, 1), jnp.float32)],
    )(x_n0, W_lin, W_conv, b_conv.reshape(1, 2), w_ar)

    dst = edge_index_n0_to_n0[0].astype(jnp.int32)
    src = edge_index_n0_to_n0[1].astype(jnp.int32)
    e = src.shape[0]
    epad = 16 * EPT
    npad = epad - e
    pad_iota = jnp.arange(npad, dtype=jnp.int32)
    src_f = jnp.concatenate([src, pad_iota % jnp.int32(N)])
    dst_f = jnp.concatenate(
        [dst, jnp.int32(N) + pad_iota % jnp.int32(2 * HALF - N)])
    spread = jnp.int32(HALF) + (dst_f % jnp.int32(NR - HALF))
    loc0 = jnp.where(dst_f < HALF, dst_f, spread)
    loc1 = jnp.where(dst_f >= HALF, dst_f - jnp.int32(HALF), spread)
    src_p = src_f.reshape(16, NB, SB, CH)
    dstl = jnp.stack([loc0, loc1]).reshape(2, 16, NB, SB, CH)
    gflat = jnp.concatenate(
        [g.reshape(N), jnp.zeros((GDR * CH - N,), jnp.float32)]
    ).reshape(GDR, CH)
    zrow = jnp.zeros((CH, CH), jnp.float32)
    iden = jnp.arange(DDR, dtype=jnp.int32).reshape(1, DDR)

    parts, denp = _edge_kernel(pg, src_p, dstl, gflat, zrow, iden)

    den = jnp.concatenate(
        [denp[0].reshape(NR)[:HALF], denp[1].reshape(NR)[:N - HALF]]
    ).reshape(N, 1)

    out = pl.pallas_call(
        _combine_body,
        out_shape=jax.ShapeDtypeStruct((N, D), jnp.float32),
    )(parts, hb1, b0, den)
    return out


# speedup vs baseline: 2.0296x; 1.3654x over previous
import functools

import jax
import jax.numpy as jnp
from jax import lax
from jax.experimental import pallas as pl
from jax.experimental.pallas import tpu as pltpu
from jax.experimental.pallas import tpu_sc as plsc

N = 10000
D = 128
HALF = 5120
NR = 5376
RPT = 336
EPT = 20480
CH = 128
NCH = EPT // CH
SB = 32
NB = NCH // SB
CMAX = EPT + CH
GDR = 80
DDR = 48


def _prep_body(x_ref, wlin_ref, wconv_ref, bconv_ref, war_ref,
               pg_ref, hb1_ref, b0_ref, g_ref):
    x = x_ref[...]
    h = lax.dot_general(x, wlin_ref[...], (((1,), (1,)), ((), ())),
                        preferred_element_type=jnp.float32)
    sr = lax.dot_general(h, war_ref[...], (((1,), (1,)), ((), ())),
                         preferred_element_type=jnp.float32)
    g = jnp.exp(sr - jnp.max(sr))
    logits = lax.dot_general(x, wconv_ref[...], (((1,), (1,)), ((), ())),
                             preferred_element_type=jnp.float32)
    logits = logits + bconv_ref[...]
    m = jnp.max(logits, axis=1, keepdims=True)
    eb = jnp.exp(logits - m)
    beta = eb / jnp.sum(eb, axis=1, keepdims=True)
    pg_ref[...] = h * g
    hb1_ref[...] = h * beta[:, 1:2]
    b0_ref[...] = beta[:, 0:1]
    g_ref[...] = g


def _combine_body(parts_ref, hb1_ref, b0_ref, den_ref, out_ref):
    agg = jnp.concatenate(
        [parts_ref[0][:HALF, :], parts_ref[1][:N - HALF, :]], axis=0)
    scale = b0_ref[...] / (den_ref[...] + 1e-30)
    out_ref[...] = agg * scale + hb1_ref[...]


_edge_mesh = plsc.VectorSubcoreMesh(core_axis_name="c", subcore_axis_name="s")


def _den_update(den_ref, d16, gv):
    io16 = lax.broadcasted_iota(jnp.int32, (16,), 0)
    k, v = plsc.sort_key_val(d16, gv)
    csum = plsc.cumsum(v)
    kprev = jnp.take_along_axis(k, jnp.maximum(io16 - 1, 0), axis=0)
    starts = jnp.logical_or(k != kprev, io16 == 0)
    knext = jnp.take_along_axis(k, jnp.minimum(io16 + 1, 15), axis=0)
    ends = jnp.logical_or(k != knext, io16 == 15)
    sidx = plsc.cummax(jnp.where(starts, io16, 0))
    cz = jnp.where(io16 == 0, 0.0,
                   jnp.take_along_axis(csum, jnp.maximum(io16 - 1, 0), axis=0))
    pc = jnp.take_along_axis(cz, sidx, axis=0)
    plsc.addupdate_scatter(den_ref,
                           [lax.shift_right_logical(k, 7),
                            lax.bitwise_and(k, 127)], csum - pc, mask=ends)


@functools.partial(
    pl.kernel,
    mesh=_edge_mesh,
    compiler_params=pltpu.CompilerParams(needs_layout_passes=False),
    out_type=[jax.ShapeDtypeStruct((2, NR, D), jnp.float32),
              jax.ShapeDtypeStruct((2, DDR, CH), jnp.float32)],
    scratch_types=[
        pltpu.VMEM((2, SB, CH), jnp.int32),
        pltpu.VMEM((2, SB, CH), jnp.int32),
        pltpu.VMEM((CMAX,), jnp.int32),
        pltpu.VMEM((2, CH), jnp.int32),
        pltpu.VMEM((2, CH), jnp.int32),
        pltpu.VMEM((2, CH, D), jnp.float32),
        pltpu.VMEM((GDR, CH), jnp.float32),
        pltpu.VMEM((DDR, CH), jnp.float32),
        pltpu.VMEM((1, DDR), jnp.int32),
        pltpu.VMEM_SHARED((NR, D), jnp.float32),
        pltpu.VMEM_SHARED((DDR, CH), jnp.float32),
        pltpu.SemaphoreType.DMA,
        pltpu.SemaphoreType.DMA,
        pltpu.SemaphoreType.DMA,
    ],
)
def _edge_kernel(pg_hbm, src_hbm, dstl_hbm, g_hbm, zrow_hbm, iden_hbm,
                 out_hbm, outden_hbm,
                 src_v, dst_v, cpk_v, sidx_v, didx_v, rows_v, g_v, den_v,
                 iden_v, accum, den_sh, sem_a, sem_s, sem_i):
    c = lax.axis_index("c")
    s = lax.axis_index("s")

    pltpu.sync_copy(src_hbm.at[s].at[0], src_v.at[0])
    pltpu.sync_copy(dstl_hbm.at[s].at[0], dst_v.at[0])
    pltpu.sync_copy(g_hbm, g_v)
    pltpu.sync_copy(iden_hbm, iden_v)

    pltpu.sync_copy(zrow_hbm.at[pl.ds(0, DDR)], den_v)
    pltpu.sync_copy(zrow_hbm, rows_v.at[0])
    for r0, rn in ((0, CH), (CH, CH), (2 * CH, RPT - 2 * CH)):
        pltpu.sync_copy(rows_v.at[0].at[pl.ds(0, rn)],
                        accum.at[pl.ds(s * RPT + r0, rn)])

    @pl.when(s == 0)
    def _():
        pltpu.sync_copy(den_v, den_sh)

    plsc.subcore_barrier()

    io16 = lax.broadcasted_iota(jnp.int32, (16,), 0)
    off = jnp.int32(0)
    for b in range(NB):
        sv = src_v.at[b % 2]
        dv = dst_v.at[b % 2]
        if b >= 1:
            pltpu.make_async_copy(src_hbm.at[s].at[b], sv, sem_i).wait()
            pltpu.make_async_copy(dstl_hbm.at[s].at[b], dv, sem_i).wait()
        if b + 1 < NB:
            nxt = (b + 1) % 2
            pltpu.async_copy(src_hbm.at[s].at[b + 1], src_v.at[nxt], sem_i)
            pltpu.async_copy(dstl_hbm.at[s].at[b + 1], dst_v.at[nxt], sem_i)

        def scan_body(j, off, sv=sv, dv=dv):
            for t in range(CH // 16):
                s16 = sv[j, pl.ds(t * 16, 16)]
                d16 = dv[j, pl.ds(t * 16, 16)] - c * HALF
                m = jnp.logical_and(d16 >= 0, d16 < HALF)
                pk = lax.bitwise_or(lax.shift_left(d16, 16), s16)
                plsc.store_compressed(cpk_v.at[pl.ds(off, 16)], pk, mask=m)
                cnt = jnp.max(plsc.all_reduce_population_count(m))
                off = off + cnt
            return off

        off = lax.fori_loop(0, SB, scan_body, off)

    for t in range(CH // 16):
        cpk_v[pl.ds(off + t * 16, 16)] = lax.shift_left(
            jnp.int32(HALF) + t * 16 + io16, 16)

    nch = jnp.maximum(lax.shift_right_logical(off + (CH - 1), 7), 1)

    def _unpack(j, p):
        for t in range(CH // 16):
            pk = cpk_v[pl.ds(j * CH + t * 16, 16)]
            s16 = lax.bitwise_and(pk, 0xFFFF)
            d16 = lax.shift_right_logical(pk, 16)
            sidx_v[p, pl.ds(t * 16, 16)] = s16
            didx_v[p, pl.ds(t * 16, 16)] = d16
            gv = plsc.load_gather(g_v, [lax.shift_right_logical(s16, 7),
                                        lax.bitwise_and(s16, 127)])
            _den_update(den_v, d16, gv)

    _unpack(0, 0)
    pltpu.async_copy(pg_hbm.at[sidx_v.at[0]], rows_v.at[0], sem_a)

    def row_body(j, carry):
        p = lax.bitwise_and(j, 1)
        pn = lax.bitwise_and(j + 1, 1)

        @pl.when(j >= 1)
        def _():
            pltpu.make_async_copy(rows_v.at[pn], accum.at[didx_v.at[pn]],
                                  sem_s).wait()

        @pl.when(j + 1 < nch)
        def _():
            _unpack(j + 1, pn)
            pltpu.async_copy(pg_hbm.at[sidx_v.at[pn]], rows_v.at[pn], sem_a)

        pltpu.make_async_copy(pg_hbm.at[sidx_v.at[p]], rows_v.at[p],
                              sem_a).wait()
        pltpu.async_copy(rows_v.at[p], accum.at[didx_v.at[p]], sem_s,
                         add=True)
        return carry

    lax.fori_loop(0, nch, row_body, 0)
    pl_last = lax.bitwise_and(nch - 1, 1)
    pltpu.make_async_copy(rows_v.at[pl_last], accum.at[didx_v.at[pl_last]],
                          sem_s).wait()

    pltpu.sync_copy(den_v, den_sh.at[iden_v.at[0]], add=True)
    plsc.subcore_barrier()

    for r0, rn in ((0, CH), (CH, CH), (2 * CH, RPT - 2 * CH)):
        ra = s * RPT + r0
        pltpu.sync_copy(accum.at[pl.ds(ra, rn)],
                        rows_v.at[0].at[pl.ds(0, rn)])
        pltpu.sync_copy(rows_v.at[0].at[pl.ds(0, rn)],
                        out_hbm.at[c].at[pl.ds(ra, rn)])

    @pl.when(s == 0)
    def _():
        pltpu.sync_copy(den_sh, den_v)
        pltpu.sync_copy(den_v, outden_hbm.at[c])


def kernel(x_n0, x_index_n0, edge_index_n0_to_n0, W_lin, W_conv, b_conv,
           w_al, b_al, w_ar, b_ar):
    del x_index_n0, w_al, b_al, b_ar
    pg, hb1, b0, g = pl.pallas_call(
        _prep_body,
        out_shape=[jax.ShapeDtypeStruct((N, D), jnp.float32),
                   jax.ShapeDtypeStruct((N, D), jnp.float32),
                   jax.ShapeDtypeStruct((N, 1), jnp.float32),
                   jax.ShapeDtypeStruct((N, 1), jnp.float32)],
    )(x_n0, W_lin, W_conv, b_conv.reshape(1, 2), w_ar)

    dst = edge_index_n0_to_n0[0].astype(jnp.int32)
    src = edge_index_n0_to_n0[1].astype(jnp.int32)
    e = src.shape[0]
    epad = 16 * EPT
    npad = epad - e
    pad_iota = jnp.arange(npad, dtype=jnp.int32)
    src_f = jnp.concatenate([src, pad_iota % jnp.int32(N)])
    dst_f = jnp.concatenate(
        [dst, jnp.int32(N) + pad_iota % jnp.int32(2 * HALF - N)])
    src_p = src_f.reshape(16, NB, SB, CH)
    dstl = dst_f.reshape(16, NB, SB, CH)
    gflat = jnp.concatenate(
        [g.reshape(N), jnp.zeros((GDR * CH - N,), jnp.float32)]
    ).reshape(GDR, CH)
    zrow = jnp.zeros((CH, CH), jnp.float32)
    iden = jnp.arange(DDR, dtype=jnp.int32).reshape(1, DDR)

    parts, denp = _edge_kernel(pg, src_p, dstl, gflat, zrow, iden)

    den = jnp.concatenate(
        [denp[0].reshape(DDR * CH)[:HALF], denp[1].reshape(DDR * CH)[:N - HALF]]
    ).reshape(N, 1)

    out = pl.pallas_call(
        _combine_body,
        out_shape=jax.ShapeDtypeStruct((N, D), jnp.float32),
    )(parts, hb1, b0, den)
    return out


# speedup vs baseline: 2.0338x; 1.0021x over previous
import functools

import jax
import jax.numpy as jnp
from jax import lax
from jax.experimental import pallas as pl
from jax.experimental.pallas import tpu as pltpu
from jax.experimental.pallas import tpu_sc as plsc

N = 10000
D = 128
HALF = 5120
NR = 5376
RPT = 336
EPT = 20480
CH = 128
NCH = EPT // CH
SB = 32
NB = NCH // SB
CMAX = EPT + CH
GDR = 80
DDR = 48


def _prep_body(x_ref, wlin_ref, wconv_ref, bconv_ref, war_ref,
               pg_ref, hb1_ref, b0_ref, g_ref):
    x = x_ref[...]
    h = lax.dot_general(x, wlin_ref[...], (((1,), (1,)), ((), ())),
                        preferred_element_type=jnp.float32)
    sr = lax.dot_general(h, war_ref[...], (((1,), (1,)), ((), ())),
                         preferred_element_type=jnp.float32)
    g = jnp.exp(sr - jnp.max(sr))
    logits = lax.dot_general(x, wconv_ref[...], (((1,), (1,)), ((), ())),
                             preferred_element_type=jnp.float32)
    logits = logits + bconv_ref[...]
    m = jnp.max(logits, axis=1, keepdims=True)
    eb = jnp.exp(logits - m)
    beta = eb / jnp.sum(eb, axis=1, keepdims=True)
    pg_ref[...] = h * g
    hb1_ref[...] = h * beta[:, 1:2]
    b0_ref[...] = beta[:, 0:1]
    g_ref[...] = g


def _combine_body(parts_ref, hb1_ref, b0_ref, den_ref, out_ref):
    agg = jnp.concatenate(
        [parts_ref[0][:HALF, :], parts_ref[1][:N - HALF, :]], axis=0)
    scale = b0_ref[...] / (den_ref[...] + 1e-30)
    out_ref[...] = agg * scale + hb1_ref[...]


_edge_mesh = plsc.VectorSubcoreMesh(core_axis_name="c", subcore_axis_name="s")


def _den_update(den_ref, d16, gv):
    io16 = lax.broadcasted_iota(jnp.int32, (16,), 0)
    k, v = plsc.sort_key_val(d16, gv)
    csum = plsc.cumsum(v)
    kprev = jnp.take_along_axis(k, jnp.maximum(io16 - 1, 0), axis=0)
    starts = jnp.logical_or(k != kprev, io16 == 0)
    knext = jnp.take_along_axis(k, jnp.minimum(io16 + 1, 15), axis=0)
    ends = jnp.logical_or(k != knext, io16 == 15)
    sidx = plsc.cummax(jnp.where(starts, io16, 0))
    cz = jnp.where(io16 == 0, 0.0,
                   jnp.take_along_axis(csum, jnp.maximum(io16 - 1, 0), axis=0))
    pc = jnp.take_along_axis(cz, sidx, axis=0)
    plsc.addupdate_scatter(den_ref,
                           [lax.shift_right_logical(k, 7),
                            lax.bitwise_and(k, 127)], csum - pc, mask=ends)


@functools.partial(
    pl.kernel,
    mesh=_edge_mesh,
    compiler_params=pltpu.CompilerParams(needs_layout_passes=False),
    out_type=[jax.ShapeDtypeStruct((2, NR, D), jnp.float32),
              jax.ShapeDtypeStruct((2, DDR, CH), jnp.float32)],
    scratch_types=[
        pltpu.VMEM((2, SB, CH), jnp.int32),
        pltpu.VMEM((CMAX,), jnp.int32),
        pltpu.VMEM((2, CH), jnp.int32),
        pltpu.VMEM((2, CH), jnp.int32),
        pltpu.VMEM((2, CH, D), jnp.float32),
        pltpu.VMEM((GDR, CH), jnp.float32),
        pltpu.VMEM((DDR, CH), jnp.float32),
        pltpu.VMEM((1, DDR), jnp.int32),
        pltpu.VMEM_SHARED((NR, D), jnp.float32),
        pltpu.VMEM_SHARED((DDR, CH), jnp.float32),
        pltpu.SemaphoreType.DMA,
        pltpu.SemaphoreType.DMA,
        pltpu.SemaphoreType.DMA,
    ],
)
def _edge_kernel(pg_hbm, pk_hbm, g_hbm, zrow_hbm, iden_hbm,
                 out_hbm, outden_hbm,
                 pk_v, cpk_v, sidx_v, didx_v, rows_v, g_v, den_v,
                 iden_v, accum, den_sh, sem_a, sem_s, sem_i):
    c = lax.axis_index("c")
    s = lax.axis_index("s")

    pltpu.sync_copy(pk_hbm.at[s].at[0], pk_v.at[0])
    pltpu.sync_copy(g_hbm, g_v)
    pltpu.sync_copy(iden_hbm, iden_v)

    pltpu.sync_copy(zrow_hbm.at[pl.ds(0, DDR)], den_v)
    pltpu.sync_copy(zrow_hbm, rows_v.at[0])
    for r0, rn in ((0, CH), (CH, CH), (2 * CH, RPT - 2 * CH)):
        pltpu.sync_copy(rows_v.at[0].at[pl.ds(0, rn)],
                        accum.at[pl.ds(s * RPT + r0, rn)])

    @pl.when(s == 0)
    def _():
        pltpu.sync_copy(den_v, den_sh)

    plsc.subcore_barrier()

    io16 = lax.broadcasted_iota(jnp.int32, (16,), 0)
    off = jnp.int32(0)
    cshift = lax.shift_left(c * HALF, 16)
    for b in range(NB):
        pv = pk_v.at[b % 2]
        if b >= 1:
            pltpu.make_async_copy(pk_hbm.at[s].at[b], pv, sem_i).wait()
        if b + 1 < NB:
            pltpu.async_copy(pk_hbm.at[s].at[b + 1], pk_v.at[(b + 1) % 2],
                             sem_i)

        def scan_body(j, off, pv=pv):
            for t in range(CH // 16):
                pkv = pv[j, pl.ds(t * 16, 16)]
                d16 = lax.shift_right_logical(pkv, 16) - c * HALF
                m = jnp.logical_and(d16 >= 0, d16 < HALF)
                plsc.store_compressed(cpk_v.at[pl.ds(off, 16)], pkv - cshift,
                                      mask=m)
                cnt = jnp.max(plsc.all_reduce_population_count(m))
                off = off + cnt
            return off

        off = lax.fori_loop(0, SB, scan_body, off)

    for t in range(CH // 16):
        cpk_v[pl.ds(off + t * 16, 16)] = lax.shift_left(
            jnp.int32(HALF) + t * 16 + io16, 16)

    nch = jnp.maximum(lax.shift_right_logical(off + (CH - 1), 7), 1)

    def _unpack(j, p):
        for t in range(CH // 16):
            pk = cpk_v[pl.ds(j * CH + t * 16, 16)]
            s16 = lax.bitwise_and(pk, 0xFFFF)
            d16 = lax.shift_right_logical(pk, 16)
            sidx_v[p, pl.ds(t * 16, 16)] = s16
            didx_v[p, pl.ds(t * 16, 16)] = d16
            gv = plsc.load_gather(g_v, [lax.shift_right_logical(s16, 7),
                                        lax.bitwise_and(s16, 127)])
            _den_update(den_v, d16, gv)

    _unpack(0, 0)
    pltpu.async_copy(pg_hbm.at[sidx_v.at[0]], rows_v.at[0], sem_a)

    def row_body(j, carry):
        p = lax.bitwise_and(j, 1)
        pn = lax.bitwise_and(j + 1, 1)

        @pl.when(j >= 1)
        def _():
            pltpu.make_async_copy(rows_v.at[pn], accum.at[didx_v.at[pn]],
                                  sem_s).wait()

        @pl.when(j + 1 < nch)
        def _():
            _unpack(j + 1, pn)
            pltpu.async_copy(pg_hbm.at[sidx_v.at[pn]], rows_v.at[pn], sem_a)

        pltpu.make_async_copy(pg_hbm.at[sidx_v.at[p]], rows_v.at[p],
                              sem_a).wait()
        pltpu.async_copy(rows_v.at[p], accum.at[didx_v.at[p]], sem_s,
                         add=True)
        return carry

    lax.fori_loop(0, nch, row_body, 0)
    pl_last = lax.bitwise_and(nch - 1, 1)
    pltpu.make_async_copy(rows_v.at[pl_last], accum.at[didx_v.at[pl_last]],
                          sem_s).wait()

    pltpu.sync_copy(den_v, den_sh.at[iden_v.at[0]], add=True)
    plsc.subcore_barrier()

    for r0, rn in ((0, CH), (CH, CH), (2 * CH, RPT - 2 * CH)):
        ra = s * RPT + r0
        pltpu.sync_copy(accum.at[pl.ds(ra, rn)],
                        rows_v.at[0].at[pl.ds(0, rn)])
        pltpu.sync_copy(rows_v.at[0].at[pl.ds(0, rn)],
                        out_hbm.at[c].at[pl.ds(ra, rn)])

    @pl.when(s == 0)
    def _():
        pltpu.sync_copy(den_sh, den_v)
        pltpu.sync_copy(den_v, outden_hbm.at[c])


def kernel(x_n0, x_index_n0, edge_index_n0_to_n0, W_lin, W_conv, b_conv,
           w_al, b_al, w_ar, b_ar):
    del x_index_n0, w_al, b_al, b_ar
    pg, hb1, b0, g = pl.pallas_call(
        _prep_body,
        out_shape=[jax.ShapeDtypeStruct((N, D), jnp.float32),
                   jax.ShapeDtypeStruct((N, D), jnp.float32),
                   jax.ShapeDtypeStruct((N, 1), jnp.float32),
                   jax.ShapeDtypeStruct((N, 1), jnp.float32)],
    )(x_n0, W_lin, W_conv, b_conv.reshape(1, 2), w_ar)

    dst = edge_index_n0_to_n0[0].astype(jnp.int32)
    src = edge_index_n0_to_n0[1].astype(jnp.int32)
    e = src.shape[0]
    epad = 16 * EPT
    npad = epad - e
    pad_iota = jnp.arange(npad, dtype=jnp.int32)
    src_f = jnp.concatenate([src, pad_iota % jnp.int32(N)])
    dst_f = jnp.concatenate(
        [dst, jnp.int32(N) + pad_iota % jnp.int32(2 * HALF - N)])
    pk_p = jnp.bitwise_or(jnp.left_shift(dst_f, 16),
                          src_f).reshape(16, NB, SB, CH)
    gflat = jnp.concatenate(
        [g.reshape(N), jnp.zeros((GDR * CH - N,), jnp.float32)]
    ).reshape(GDR, CH)
    zrow = jnp.zeros((CH, CH), jnp.float32)
    iden = jnp.arange(DDR, dtype=jnp.int32).reshape(1, DDR)

    parts, denp = _edge_kernel(pg, pk_p, gflat, zrow, iden)

    den = jnp.concatenate(
        [denp[0].reshape(DDR * CH)[:HALF], denp[1].reshape(DDR * CH)[:N - HALF]]
    ).reshape(N, 1)

    out = pl.pallas_call(
        _combine_body,
        out_shape=jax.ShapeDtypeStruct((N, D), jnp.float32),
    )(parts, hb1, b0, den)
    return out
